# Initial kernel scaffold; baseline (speedup 1.0000x reference)
import jax
import jax.numpy as jnp
from jax.experimental import pallas as pl


def kernel(x, Wg, bg, W1, b1, W2, b2, W3, b3):
    raise NotImplementedError("write your pallas kernel here")


# speedup vs baseline: 1.5693x; 1.5693x over previous
import functools

import jax
import jax.numpy as jnp
from jax import lax
from jax.experimental import pallas as pl
from jax.experimental.pallas import tpu as pltpu
from jax.experimental.pallas import tpu_sc as plsc

NC = 2
NS = 16
NW = NC * NS

TILE = 128


def _gating_body(x_ref, wg_ref, bg_ref, i0_ref, i1_ref, g0_ref, g1_ref):
    x = x_ref[...]
    logits = jnp.dot(x, wg_ref[...], preferred_element_type=jnp.float32)
    logits = logits + bg_ref[...]
    gb, e = logits.shape
    iota = lax.broadcasted_iota(jnp.int32, (gb, e), 1)
    m1 = jnp.max(logits, axis=1, keepdims=True)
    i1 = jnp.min(jnp.where(logits == m1, iota, e), axis=1, keepdims=True)
    l2 = jnp.where(iota == i1, -jnp.inf, logits)
    m2 = jnp.max(l2, axis=1, keepdims=True)
    i2 = jnp.min(jnp.where(l2 == m2, iota, e), axis=1, keepdims=True)
    e2 = jnp.exp(m2 - m1)
    s = 1.0 + e2
    i0_ref[...] = i1
    i1_ref[...] = i2
    g0_ref[...] = 1.0 / s
    g1_ref[...] = e2 / s


def _gating(x, Wg, bg):
    n, d = x.shape
    e = Wg.shape[1]
    gb = 512
    return pl.pallas_call(
        _gating_body,
        grid=(n // gb,),
        in_specs=[
            pl.BlockSpec((gb, d), lambda i: (i, 0)),
            pl.BlockSpec((d, e), lambda i: (0, 0)),
            pl.BlockSpec((1, e), lambda i: (0, 0)),
        ],
        out_specs=[
            pl.BlockSpec((gb, 1), lambda i: (i, 0)),
            pl.BlockSpec((gb, 1), lambda i: (i, 0)),
            pl.BlockSpec((gb, 1), lambda i: (i, 0)),
            pl.BlockSpec((gb, 1), lambda i: (i, 0)),
        ],
        out_shape=[
            jax.ShapeDtypeStruct((n, 1), jnp.int32),
            jax.ShapeDtypeStruct((n, 1), jnp.int32),
            jax.ShapeDtypeStruct((n, 1), jnp.float32),
            jax.ShapeDtypeStruct((n, 1), jnp.float32),
        ],
    )(x, Wg, bg.reshape(1, e))


def _make_gather(n_rows, d, p):
    per_w = p // NW
    ch = 40
    n_ch = per_w // ch
    assert per_w % ch == 0
    mesh = plsc.VectorSubcoreMesh(
        core_axis_name="c", subcore_axis_name="s",
        num_cores=NC, num_subcores=NS)

    def body(x_hbm, rids_hbm, out_hbm, idx_v, rows_v, sem):
        wid = lax.axis_index("s") * NC + lax.axis_index("c")
        pltpu.sync_copy(rids_hbm.at[wid], idx_v)
        for c in range(n_ch):
            pltpu.async_copy(x_hbm.at[idx_v.at[c]], rows_v, sem).wait()
            pltpu.sync_copy(
                rows_v, out_hbm.at[pl.ds(wid * per_w + c * ch, ch)])

    return pl.kernel(
        body,
        out_type=jax.ShapeDtypeStruct((p, d), jnp.float32),
        mesh=mesh,
        scratch_types=[
            pltpu.VMEM((n_ch, ch), jnp.int32),
            pltpu.VMEM((ch, d), jnp.float32),
            pltpu.SemaphoreType.DMA,
        ],
    )


def _mlp_body(te_ref, xd_ref, gs_ref, w1_ref, b1_ref, w2_ref, b2_ref,
              w3_ref, b3_ref, out_ref):
    xb = xd_ref[...].astype(jnp.bfloat16)
    h = jnp.dot(xb, w1_ref[0], preferred_element_type=jnp.float32)
    h = jnp.maximum(h + b1_ref[0], 0.0).astype(jnp.bfloat16)
    h = jnp.dot(h, w2_ref[0], preferred_element_type=jnp.float32)
    h = jnp.maximum(h + b2_ref[0], 0.0).astype(jnp.bfloat16)
    o = jnp.dot(h, w3_ref[0], preferred_element_type=jnp.float32)
    out_ref[...] = (o + b3_ref[0]) * gs_ref[...]


def _mlp(te, xd, gs, W1, b1, W2, b2, W3, b3):
    p, d = xd.shape
    e, _, h = W1.shape
    o = W3.shape[2]
    b1 = b1.reshape(e, 1, h)
    b2 = b2.reshape(e, 1, h)
    b3 = b3.reshape(e, 1, o)
    nt = p // TILE
    grid_spec = pltpu.PrefetchScalarGridSpec(
        num_scalar_prefetch=1,
        grid=(nt,),
        in_specs=[
            pl.BlockSpec((TILE, d), lambda t, te: (t, 0)),
            pl.BlockSpec((TILE, 1), lambda t, te: (t, 0)),
            pl.BlockSpec((1, d, h), lambda t, te: (te[t], 0, 0)),
            pl.BlockSpec((1, 1, h), lambda t, te: (te[t], 0, 0)),
            pl.BlockSpec((1, h, h), lambda t, te: (te[t], 0, 0)),
            pl.BlockSpec((1, 1, h), lambda t, te: (te[t], 0, 0)),
            pl.BlockSpec((1, h, o), lambda t, te: (te[t], 0, 0)),
            pl.BlockSpec((1, 1, o), lambda t, te: (te[t], 0, 0)),
        ],
        out_specs=pl.BlockSpec((TILE, o), lambda t, te: (t, 0)),
    )
    return pl.pallas_call(
        _mlp_body,
        grid_spec=grid_spec,
        out_shape=jax.ShapeDtypeStruct((p, o), jnp.float32),
        compiler_params=pltpu.CompilerParams(
            dimension_semantics=("arbitrary",)),
    )(te, xd, gs, W1, b1, W2, b2, W3, b3)


def _make_combine(n, o, p):
    per_w = n // NW
    ch = 32
    n_ch = per_w // ch
    vec = 16
    mesh = plsc.VectorSubcoreMesh(
        core_axis_name="c", subcore_axis_name="s",
        num_cores=NC, num_subcores=NS)

    def body(y_hbm, d0_hbm, d1_hbm, out_hbm,
             d0_v, d1_v, buf0, buf1, sem0, sem1):
        wid = lax.axis_index("s") * NC + lax.axis_index("c")
        pltpu.sync_copy(d0_hbm.at[wid], d0_v)
        pltpu.sync_copy(d1_hbm.at[wid], d1_v)
        for c in range(n_ch):
            ca = pltpu.async_copy(y_hbm.at[d0_v.at[c]], buf0, sem0)
            cb = pltpu.async_copy(y_hbm.at[d1_v.at[c]], buf1, sem1)
            ca.wait()
            cb.wait()

            def row_add(r, _):
                for j in range(o // vec):
                    sl = pl.ds(j * vec, vec)
                    buf0[r, sl] = buf0[r, sl] + buf1[r, sl]
                return 0

            lax.fori_loop(0, ch, row_add, 0)
            pltpu.sync_copy(
                buf0, out_hbm.at[pl.ds(wid * per_w + c * ch, ch)])

    return pl.kernel(
        body,
        out_type=jax.ShapeDtypeStruct((n, o), jnp.float32),
        mesh=mesh,
        scratch_types=[
            pltpu.VMEM((n_ch, ch), jnp.int32),
            pltpu.VMEM((n_ch, ch), jnp.int32),
            pltpu.VMEM((ch, o), jnp.float32),
            pltpu.VMEM((ch, o), jnp.float32),
            pltpu.SemaphoreType.DMA,
            pltpu.SemaphoreType.DMA,
        ],
    )


def kernel(x, Wg, bg, W1, b1, W2, b2, W3, b3):
    n, d = x.shape
    e = Wg.shape[1]
    k = 2
    nk = n * k
    p = nk + e * TILE
    o = W3.shape[2]

    i0, i1, g0, g1 = _gating(x, Wg, bg)

    ef = jnp.concatenate([i0, i1], axis=1).reshape(-1)
    gf = jnp.concatenate([g0, g1], axis=1).reshape(-1)
    onehot = (ef[:, None] == jnp.arange(e)[None, :]).astype(jnp.int32)
    rank = jnp.sum((jnp.cumsum(onehot, axis=0) - onehot) * onehot, axis=1)
    counts = jnp.sum(onehot, axis=0)
    tiles_per_e = (counts + TILE - 1) // TILE
    tile_bounds = jnp.cumsum(tiles_per_e)
    astart = (tile_bounds - tiles_per_e) * TILE
    dest = astart[ef] + rank
    row_ids = jnp.zeros((p,), jnp.int32).at[dest].set(
        jnp.arange(nk, dtype=jnp.int32) // k)
    gs = jnp.zeros((p,), jnp.float32).at[dest].set(gf).reshape(p, 1)
    te = jnp.minimum(
        jnp.searchsorted(tile_bounds, jnp.arange(p // TILE), side="right"),
        e - 1).astype(jnp.int32)

    per_w = p // NW
    rids = row_ids.reshape(NW, per_w // 40, 40)
    xd = _make_gather(n, d, p)(x, rids)

    bf = jnp.bfloat16
    y = _mlp(te, xd, gs, W1.astype(bf), b1, W2.astype(bf), b2,
             W3.astype(bf), b3)

    dp = dest.reshape(n, k)
    d0 = dp[:, 0].reshape(NW, (n // NW) // 32, 32)
    d1 = dp[:, 1].reshape(NW, (n // NW) // 32, 32)
    return _make_combine(n, o, p)(y, d0, d1)
